# baseline (speedup 1.0000x reference)
import functools

import jax
import jax.numpy as jnp
from jax.experimental import pallas as pl

_N = 50000
_E = 800000
_H = 128
_L = 15
_NT = 1000
_ET = 2000


def _layer_kernel(h_ref, agg_ref, wn_ref, wr_ref, b_ref, m_ref, s_ref, q_ref):
    m = (jnp.dot(agg_ref[...], wn_ref[...], preferred_element_type=jnp.float32)
         + jnp.dot(h_ref[...], wr_ref[...], preferred_element_type=jnp.float32)
         + b_ref[...])
    m_ref[...] = m

    @pl.when(pl.program_id(0) == 0)
    def _init():
        s_ref[...] = jnp.zeros_like(s_ref)
        q_ref[...] = jnp.zeros_like(q_ref)

    s_ref[...] += jnp.sum(m, axis=0, keepdims=True)
    q_ref[...] += jnp.sum(m * m, axis=0, keepdims=True)


def _apply_kernel(h_ref, m_ref, sc_ref, sh_ref, o_ref):
    o_ref[...] = h_ref[...] + jnp.maximum(
        m_ref[...] * sc_ref[...] + sh_ref[...], 0.0)


def _input_fc_kernel(x_ref, w_ref, o_ref):
    x = x_ref[...]
    o_ref[...] = (x[:, 0:1] * w_ref[0:1, :] + x[:, 1:2] * w_ref[1:2, :])


def _edge_mlp_kernel(es_ref, ed_ref, w1a_ref, w1b_ref, b1_ref, w2_ref,
                     b2_ref, w3_ref, b3_ref, o_ref):
    z = jnp.maximum(
        jnp.dot(es_ref[...], w1a_ref[...], preferred_element_type=jnp.float32)
        + jnp.dot(ed_ref[...], w1b_ref[...], preferred_element_type=jnp.float32)
        + b1_ref[...], 0.0)
    z = jnp.maximum(
        jnp.dot(z, w2_ref[...], preferred_element_type=jnp.float32)
        + b2_ref[...], 0.0)
    t = jnp.dot(z, w3_ref[...], preferred_element_type=jnp.float32)
    o_ref[...] = jax.nn.sigmoid(t[:, 0:1] + b3_ref[...])


def _node_tiles(i):
    return (i, 0)


def _resident(i):
    return (0, 0)


_layer_call = pl.pallas_call(
    _layer_kernel,
    grid=(_N // _NT,),
    in_specs=[
        pl.BlockSpec((_NT, _H), _node_tiles),
        pl.BlockSpec((_NT, _H), _node_tiles),
        pl.BlockSpec((_H, _H), _resident),
        pl.BlockSpec((_H, _H), _resident),
        pl.BlockSpec((1, _H), _resident),
    ],
    out_specs=[
        pl.BlockSpec((_NT, _H), _node_tiles),
        pl.BlockSpec((1, _H), _resident),
        pl.BlockSpec((1, _H), _resident),
    ],
    out_shape=[
        jax.ShapeDtypeStruct((_N, _H), jnp.float32),
        jax.ShapeDtypeStruct((1, _H), jnp.float32),
        jax.ShapeDtypeStruct((1, _H), jnp.float32),
    ],
)

_apply_call = pl.pallas_call(
    _apply_kernel,
    grid=(_N // _NT,),
    in_specs=[
        pl.BlockSpec((_NT, _H), _node_tiles),
        pl.BlockSpec((_NT, _H), _node_tiles),
        pl.BlockSpec((1, _H), _resident),
        pl.BlockSpec((1, _H), _resident),
    ],
    out_specs=pl.BlockSpec((_NT, _H), _node_tiles),
    out_shape=jax.ShapeDtypeStruct((_N, _H), jnp.float32),
)

_input_fc_call = pl.pallas_call(
    _input_fc_kernel,
    grid=(_N // _NT,),
    in_specs=[
        pl.BlockSpec((_NT, 2), _node_tiles),
        pl.BlockSpec((2, _H), _resident),
    ],
    out_specs=pl.BlockSpec((_NT, _H), _node_tiles),
    out_shape=jax.ShapeDtypeStruct((_N, _H), jnp.float32),
)

_edge_mlp_call = pl.pallas_call(
    _edge_mlp_kernel,
    grid=(_E // _ET,),
    in_specs=[
        pl.BlockSpec((_ET, _H), _node_tiles),
        pl.BlockSpec((_ET, _H), _node_tiles),
        pl.BlockSpec((_H, _H), _resident),
        pl.BlockSpec((_H, _H), _resident),
        pl.BlockSpec((1, _H), _resident),
        pl.BlockSpec((_H, _H), _resident),
        pl.BlockSpec((1, _H), _resident),
        pl.BlockSpec((_H, _H), _resident),
        pl.BlockSpec((1, 1), _resident),
    ],
    out_specs=pl.BlockSpec((_ET, 1), _node_tiles),
    out_shape=jax.ShapeDtypeStruct((_E, 1), jnp.float32),
)


@functools.partial(jax.jit)
def _run(x, edge_index, W_in, W_root, W_nbr, b_nbr, gamma, beta,
         W1, b1, W2, b2, W3, b3):
    row = edge_index[0]
    col = edge_index[1]
    h = _input_fc_call(x, W_in)
    inv_n = jnp.float32(1.0 / _N)
    for i in range(_L):
        msg = jnp.take(h, row, axis=0)
        agg = jax.ops.segment_sum(msg, col, num_segments=_N)
        m, s, q = _layer_call(h, agg, W_nbr[i], W_root[i],
                              b_nbr[i].reshape(1, _H))
        mu = s * inv_n
        var = q * inv_n - mu * mu
        scale = gamma[i].reshape(1, _H) * jax.lax.rsqrt(var + 1e-5)
        shift = beta[i].reshape(1, _H) - mu * scale
        h = _apply_call(h, m, scale, shift)
    e_src = jnp.take(h, row, axis=0)
    e_dst = jnp.take(h, col, axis=0)
    w3p = jnp.zeros((_H, _H), jnp.float32).at[:, 0:1].set(W3)
    out = _edge_mlp_call(e_src, e_dst, W1[:_H], W1[_H:],
                         b1.reshape(1, _H), W2, b2.reshape(1, _H),
                         w3p, b3.reshape(1, 1))
    return out


def kernel(x, edge_index, W_in, W_root, W_nbr, b_nbr, gamma, beta,
           W1, b1, W2, b2, W3, b3):
    return _run(x, edge_index, W_in, W_root, W_nbr, b_nbr, gamma, beta,
                W1, b1, W2, b2, W3, b3)
